# baseline (speedup 1.0000x reference)
import jax
import jax.numpy as jnp
from jax import lax
from jax.experimental import pallas as pl
from jax.experimental.pallas import tpu as pltpu
from jax.experimental.pallas import tpu_sc as plsc

_N = 10000
_D = 128
_E = 320000
_STEPS = 16
_HH = 64
_NS = 16
_NC = 2
_SCH = 128
_GCH = 256
_EPT = _E // _NS
_NGCH = 81
_NSCH = 2 * _NGCH
_EPADN = _NGCH * _GCH - _EPT
_TRASH = 96
_NPAD = _N + _TRASH
_RPT = _N // _NS
_RCH = 125
_NRCH = _RPT // _RCH
_NRING = 27
_ZPT = _NPAD // _NS
_ZSZ = (128, 128, 128, 128, 119)
_UA = (0, 375)
_UW = (125, 500)
_UY = (250, 625)


def _body(xs, srcs, dsts, ones_h, zeros_h, yh, wv,
          acc, idx_s, idx_d, bufs, gsem, ssem):
    c = lax.axis_index("c")
    s = lax.axis_index("s")
    yh_c = yh.at[c]
    wv_c = wv.at[c]
    xs_c = xs.at[c]

    pltpu.sync_copy(srcs.at[s], idx_s)
    pltpu.sync_copy(dsts.at[s], idx_d)
    pltpu.sync_copy(ones_h, bufs.at[pl.ds(0, _SCH)])

    z0 = s * _ZPT
    zoff = 0
    for zs in _ZSZ:
        pltpu.sync_copy(zeros_h.at[pl.ds(0, zs)],
                        acc.at[pl.ds(z0 + zoff, zs)])
        zoff += zs
    plsc.subcore_barrier()

    def deg_body(i, carry):
        sd = []
        for q in range(6):
            sd.append(pltpu.async_copy(bufs.at[pl.ds(0, _SCH)],
                                       acc.at[idx_d.at[6 * i + q]],
                                       ssem.at[q], add=True))
        for d in sd:
            d.wait()
        return carry
    lax.fori_loop(0, _NSCH // 6, deg_body, None)
    plsc.subcore_barrier()

    r0 = s * _RPT
    for k in range(_NRCH):
        rows = pl.ds(r0 + k * _RCH, _RCH)
        pltpu.sync_copy(acc.at[rows], bufs.at[pl.ds(0, _RCH)])

        def init_row(r, carry):
            for jj in range(_HH // 16):
                sl = pl.ds(jj * 16, 16)
                bufs[r, sl] = 0.5 / (1.0 + bufs[r, sl])
            return carry
        lax.fori_loop(0, _RCH, init_row, None)

        pltpu.sync_copy(bufs.at[pl.ds(0, _RCH)], wv_c.at[rows])
        pltpu.sync_copy(xs_c.at[rows], bufs.at[pl.ds(_RCH, _RCH)])
        pltpu.sync_copy(bufs.at[pl.ds(_RCH, _RCH)], yh_c.at[rows])
        pltpu.sync_copy(bufs.at[pl.ds(_RCH, _RCH)], acc.at[rows])
    plsc.subcore_barrier()

    def step(t, carry):
        def ring(i, rcarry):
            gd = []
            for q in range(3):
                @pl.when(i > 0)
                def _wait_prev(q=q):
                    for h in range(2):
                        pltpu.make_async_copy(
                            bufs.at[pl.ds(q * _GCH + h * _SCH, _SCH)],
                            acc.at[pl.ds(0, _SCH)],
                            ssem.at[2 * q + h]).wait()
                j = 3 * i + q
                gd.append(pltpu.async_copy(
                    yh_c.at[idx_s.at[j]],
                    bufs.at[pl.ds(q * _GCH, _GCH)], gsem.at[q]))
            for q in range(3):
                j = 3 * i + q
                gd[q].wait()
                for h in range(2):
                    pltpu.async_copy(
                        bufs.at[pl.ds(q * _GCH + h * _SCH, _SCH)],
                        acc.at[idx_d.at[2 * j + h]],
                        ssem.at[2 * q + h], add=True)
            return rcarry
        lax.fori_loop(0, _NRING, ring, None)
        for q in range(3):
            for h in range(2):
                pltpu.make_async_copy(
                    bufs.at[pl.ds(q * _GCH + h * _SCH, _SCH)],
                    acc.at[pl.ds(0, _SCH)], ssem.at[2 * q + h]).wait()
        plsc.subcore_barrier()

        def fire_loads(k, p):
            rows = pl.ds(r0 + k * _RCH, _RCH)
            pltpu.async_copy(acc.at[rows],
                             bufs.at[pl.ds(_UA[p], _RCH)], gsem.at[3 * p])
            pltpu.async_copy(wv_c.at[rows],
                             bufs.at[pl.ds(_UW[p], _RCH)],
                             gsem.at[3 * p + 1])
            pltpu.async_copy(yh_c.at[rows],
                             bufs.at[pl.ds(_UY[p], _RCH)],
                             gsem.at[3 * p + 2])

        def wait_loads(k, p):
            rows = pl.ds(r0 + k * _RCH, _RCH)
            pltpu.make_async_copy(acc.at[rows],
                                  bufs.at[pl.ds(_UA[p], _RCH)],
                                  gsem.at[3 * p]).wait()
            pltpu.make_async_copy(wv_c.at[rows],
                                  bufs.at[pl.ds(_UW[p], _RCH)],
                                  gsem.at[3 * p + 1]).wait()
            pltpu.make_async_copy(yh_c.at[rows],
                                  bufs.at[pl.ds(_UY[p], _RCH)],
                                  gsem.at[3 * p + 2]).wait()

        fire_loads(0, 0)
        for k in range(_NRCH):
            p = k % 2
            rows = pl.ds(r0 + k * _RCH, _RCH)
            wait_loads(k, p)
            if k > 0:
                pltpu.make_async_copy(xs_c.at[rows], acc.at[rows],
                                      ssem.at[6]).wait()
            pltpu.async_copy(xs_c.at[rows], acc.at[rows], ssem.at[6])
            if k + 1 < _NRCH:
                if k >= 1:
                    pltpu.make_async_copy(
                        bufs.at[pl.ds(_UY[1 - p], _RCH)],
                        yh_c.at[rows], ssem.at[7 + (1 - p)]).wait()
                fire_loads(k + 1, 1 - p)

            uy = _UY[p]
            ua = _UA[p]
            uw = _UW[p]

            def upd_row(r, ucarry):
                for jj in range(_HH // 16):
                    sl = pl.ds(jj * 16, 16)
                    bufs[uy + r, sl] = (0.5 * bufs[uy + r, sl]
                                        + bufs[uw + r, sl]
                                        * bufs[ua + r, sl])
                return ucarry
            lax.fori_loop(0, _RCH, upd_row, None)

            pltpu.async_copy(bufs.at[pl.ds(uy, _RCH)], yh_c.at[rows],
                             ssem.at[7 + p])
        pltpu.make_async_copy(bufs.at[pl.ds(_UY[0], _RCH)],
                              yh_c.at[pl.ds(r0, _RCH)], ssem.at[7]).wait()
        pltpu.make_async_copy(bufs.at[pl.ds(_UY[1], _RCH)],
                              yh_c.at[pl.ds(r0, _RCH)], ssem.at[8]).wait()
        pltpu.make_async_copy(xs_c.at[pl.ds(r0, _RCH)],
                              acc.at[pl.ds(r0, _RCH)], ssem.at[6]).wait()
        plsc.subcore_barrier()
        return carry
    lax.fori_loop(0, _STEPS, step, None)


_sc_fn = pl.kernel(
    _body,
    out_type=[
        jax.ShapeDtypeStruct((_NC, _N, _HH), jnp.float32),
        jax.ShapeDtypeStruct((_NC, _N, _HH), jnp.float32),
    ],
    mesh=plsc.VectorSubcoreMesh(core_axis_name="c", subcore_axis_name="s"),
    compiler_params=pltpu.CompilerParams(use_tc_tiling_on_sc=False),
    scratch_types=[
        pltpu.VMEM_SHARED((_NPAD, _HH), jnp.float32),
        pltpu.VMEM((_NGCH, _GCH), jnp.int32),
        pltpu.VMEM((_NSCH, _SCH), jnp.int32),
        pltpu.VMEM((3 * _GCH, _HH), jnp.float32),
        pltpu.SemaphoreType.DMA((6,)),
        pltpu.SemaphoreType.DMA((9,)),
    ],
)


def kernel(x, edge_index):
    src = edge_index[0].astype(jnp.int32)
    dst = edge_index[1].astype(jnp.int32)

    src_t = src.reshape(_NS, _EPT)
    dst_t = dst.reshape(_NS, _EPT)
    pad_i = jnp.arange(_EPADN, dtype=jnp.int32)
    pad_src = jnp.broadcast_to((pad_i * 397) % _N, (_NS, _EPADN))
    pad_dst = jnp.broadcast_to(_N + pad_i % _TRASH, (_NS, _EPADN))
    srcs = jnp.concatenate([src_t, pad_src], axis=1)
    srcs = srcs.reshape(_NS, _NGCH, _GCH)
    dsts = jnp.concatenate([dst_t, pad_dst], axis=1)
    dsts = dsts.reshape(_NS, _NSCH, _SCH)

    xs = jnp.stack([x[:, :_HH], x[:, _HH:]])
    ones_h = jnp.ones((_SCH, _HH), jnp.float32)
    zeros_h = jnp.zeros((_SCH, _HH), jnp.float32)

    yh, _wv = _sc_fn(xs, srcs, dsts, ones_h, zeros_h)
    return jnp.concatenate([yh[0], yh[1]], axis=1)


# speedup vs baseline: 1.0586x; 1.0586x over previous
import jax
import jax.numpy as jnp
from jax import lax
from jax.experimental import pallas as pl
from jax.experimental.pallas import tpu as pltpu
from jax.experimental.pallas import tpu_sc as plsc

_N = 10000
_D = 128
_E = 320000
_STEPS = 16
_HH = 64
_NS = 16
_NC = 2
_SCH = 128
_GCH = 256
_EPT = _E // _NS
_NGCH = 80
_NSCH = 2 * _NGCH
_EPADN = _NGCH * _GCH - _EPT
_TRASH = 112
_NPAD = _N + _TRASH
_RPT = _N // _NS
_RCH = 125
_NRCH = _RPT // _RCH
_G2 = _NGCH // 2
_ZPT = _NPAD // _NS
_ZSZ = (128, 128, 128, 128, 120)
_UA = (0, 375)
_UW = (125, 500)
_UY = (250, 625)


def _body(xs, srcs, dsts, ones_h, zeros_h, yh, wv,
          acc, idx_s, idx_d, bufs, gsem, ssem):
    c = lax.axis_index("c")
    s = lax.axis_index("s")
    yh_c = yh.at[c]
    wv_c = wv.at[c]
    xs_c = xs.at[c]

    pltpu.sync_copy(srcs.at[s], idx_s)
    pltpu.sync_copy(dsts.at[s], idx_d)
    pltpu.sync_copy(ones_h, bufs.at[pl.ds(0, _SCH)])

    z0 = s * _ZPT
    zoff = 0
    for zs in _ZSZ:
        pltpu.sync_copy(zeros_h.at[pl.ds(0, zs)],
                        acc.at[pl.ds(z0 + zoff, zs)])
        zoff += zs
    plsc.subcore_barrier()

    def deg_body(i, carry):
        sd = []
        for q in range(4):
            sd.append(pltpu.async_copy(bufs.at[pl.ds(0, _SCH)],
                                       acc.at[idx_d.at[4 * i + q]],
                                       ssem.at[q], add=True))
        for d in sd:
            d.wait()
        return carry
    lax.fori_loop(0, _NSCH // 4, deg_body, None)
    plsc.subcore_barrier()

    r0 = s * _RPT
    for k in range(_NRCH):
        rows = pl.ds(r0 + k * _RCH, _RCH)
        pltpu.sync_copy(acc.at[rows], bufs.at[pl.ds(0, _RCH)])

        def init_row(r, carry):
            for jj in range(_HH // 16):
                sl = pl.ds(jj * 16, 16)
                bufs[r, sl] = 0.5 / (1.0 + bufs[r, sl])
            return carry
        lax.fori_loop(0, _RCH, init_row, None)

        pltpu.sync_copy(bufs.at[pl.ds(0, _RCH)], wv_c.at[rows])
        pltpu.sync_copy(xs_c.at[rows], bufs.at[pl.ds(_RCH, _RCH)])
        pltpu.sync_copy(bufs.at[pl.ds(_RCH, _RCH)], yh_c.at[rows])
        pltpu.sync_copy(bufs.at[pl.ds(_RCH, _RCH)], acc.at[rows])
    plsc.subcore_barrier()

    def step(t, carry):
        pltpu.async_copy(yh_c.at[idx_s.at[0]],
                         bufs.at[pl.ds(0, _GCH)], gsem.at[0])

        def group2(gg, gcarry):
            g1 = pltpu.async_copy(yh_c.at[idx_s.at[2 * gg + 1]],
                                  bufs.at[pl.ds(_GCH, _GCH)], gsem.at[1])
            pltpu.make_async_copy(yh_c.at[idx_s.at[2 * gg]],
                                  bufs.at[pl.ds(0, _GCH)], gsem.at[0]).wait()
            s0 = []
            for q in range(2):
                j = 4 * gg + q
                s0.append(pltpu.async_copy(
                    bufs.at[pl.ds(q * _SCH, _SCH)],
                    acc.at[idx_d.at[j]], ssem.at[q], add=True))
            for d in s0:
                d.wait()

            @pl.when(gg < _G2 - 1)
            def _refill():
                pltpu.async_copy(yh_c.at[idx_s.at[2 * gg + 2]],
                                 bufs.at[pl.ds(0, _GCH)], gsem.at[0])

            g1.wait()
            s1 = []
            for q in range(2):
                j = 4 * gg + 2 + q
                s1.append(pltpu.async_copy(
                    bufs.at[pl.ds(_GCH + q * _SCH, _SCH)],
                    acc.at[idx_d.at[j]], ssem.at[2 + q], add=True))
            for d in s1:
                d.wait()
            return gcarry
        lax.fori_loop(0, _G2, group2, None)
        plsc.subcore_barrier()

        def fire_loads(k, p):
            rows = pl.ds(r0 + k * _RCH, _RCH)
            pltpu.async_copy(acc.at[rows],
                             bufs.at[pl.ds(_UA[p], _RCH)], gsem.at[3 * p])
            pltpu.async_copy(wv_c.at[rows],
                             bufs.at[pl.ds(_UW[p], _RCH)],
                             gsem.at[3 * p + 1])
            pltpu.async_copy(yh_c.at[rows],
                             bufs.at[pl.ds(_UY[p], _RCH)],
                             gsem.at[3 * p + 2])

        def wait_loads(k, p):
            rows = pl.ds(r0 + k * _RCH, _RCH)
            pltpu.make_async_copy(acc.at[rows],
                                  bufs.at[pl.ds(_UA[p], _RCH)],
                                  gsem.at[3 * p]).wait()
            pltpu.make_async_copy(wv_c.at[rows],
                                  bufs.at[pl.ds(_UW[p], _RCH)],
                                  gsem.at[3 * p + 1]).wait()
            pltpu.make_async_copy(yh_c.at[rows],
                                  bufs.at[pl.ds(_UY[p], _RCH)],
                                  gsem.at[3 * p + 2]).wait()

        fire_loads(0, 0)
        for k in range(_NRCH):
            p = k % 2
            rows = pl.ds(r0 + k * _RCH, _RCH)
            wait_loads(k, p)
            if k > 0:
                pltpu.make_async_copy(xs_c.at[rows], acc.at[rows],
                                      ssem.at[6]).wait()
            pltpu.async_copy(xs_c.at[rows], acc.at[rows], ssem.at[6])
            if k + 1 < _NRCH:
                if k >= 1:
                    pltpu.make_async_copy(
                        bufs.at[pl.ds(_UY[1 - p], _RCH)],
                        yh_c.at[rows], ssem.at[7 + (1 - p)]).wait()
                fire_loads(k + 1, 1 - p)

            uy = _UY[p]
            ua = _UA[p]
            uw = _UW[p]

            def upd_row(r, ucarry):
                for jj in range(_HH // 16):
                    sl = pl.ds(jj * 16, 16)
                    bufs[uy + r, sl] = (0.5 * bufs[uy + r, sl]
                                        + bufs[uw + r, sl]
                                        * bufs[ua + r, sl])
                return ucarry
            lax.fori_loop(0, _RCH, upd_row, None)

            pltpu.async_copy(bufs.at[pl.ds(uy, _RCH)], yh_c.at[rows],
                             ssem.at[7 + p])
        pltpu.make_async_copy(bufs.at[pl.ds(_UY[0], _RCH)],
                              yh_c.at[pl.ds(r0, _RCH)], ssem.at[7]).wait()
        pltpu.make_async_copy(bufs.at[pl.ds(_UY[1], _RCH)],
                              yh_c.at[pl.ds(r0, _RCH)], ssem.at[8]).wait()
        pltpu.make_async_copy(xs_c.at[pl.ds(r0, _RCH)],
                              acc.at[pl.ds(r0, _RCH)], ssem.at[6]).wait()
        plsc.subcore_barrier()
        return carry
    lax.fori_loop(0, _STEPS, step, None)


_sc_fn = pl.kernel(
    _body,
    out_type=[
        jax.ShapeDtypeStruct((_NC, _N, _HH), jnp.float32),
        jax.ShapeDtypeStruct((_NC, _N, _HH), jnp.float32),
    ],
    mesh=plsc.VectorSubcoreMesh(core_axis_name="c", subcore_axis_name="s"),
    compiler_params=pltpu.CompilerParams(use_tc_tiling_on_sc=False),
    scratch_types=[
        pltpu.VMEM_SHARED((_NPAD, _HH), jnp.float32),
        pltpu.VMEM((_NGCH, _GCH), jnp.int32),
        pltpu.VMEM((_NSCH, _SCH), jnp.int32),
        pltpu.VMEM((3 * _GCH, _HH), jnp.float32),
        pltpu.SemaphoreType.DMA((6,)),
        pltpu.SemaphoreType.DMA((9,)),
    ],
)


def kernel(x, edge_index):
    src = edge_index[0].astype(jnp.int32)
    dst = edge_index[1].astype(jnp.int32)

    src_t = src.reshape(_NS, _EPT)
    dst_t = dst.reshape(_NS, _EPT)
    pad_i = jnp.arange(_EPADN, dtype=jnp.int32)
    pad_src = jnp.broadcast_to((pad_i * 397) % _N, (_NS, _EPADN))
    pad_dst = jnp.broadcast_to(_N + pad_i % _TRASH, (_NS, _EPADN))
    srcs = jnp.concatenate([src_t, pad_src], axis=1)
    srcs = srcs.reshape(_NS, _NGCH, _GCH)
    dsts = jnp.concatenate([dst_t, pad_dst], axis=1)
    dsts = dsts.reshape(_NS, _NSCH, _SCH)

    xs = jnp.stack([x[:, :_HH], x[:, _HH:]])
    ones_h = jnp.ones((_SCH, _HH), jnp.float32)
    zeros_h = jnp.zeros((_SCH, _HH), jnp.float32)

    yh, _wv = _sc_fn(xs, srcs, dsts, ones_h, zeros_h)
    return jnp.concatenate([yh[0], yh[1]], axis=1)
